# Initial kernel scaffold; baseline (speedup 1.0000x reference)
import jax
import jax.numpy as jnp
from jax.experimental import pallas as pl


def kernel(wids, wordemb):
    raise NotImplementedError("write your pallas kernel here")


# speedup vs baseline: 3.9561x; 3.9561x over previous
import functools

import jax
import jax.numpy as jnp
from jax import lax
from jax.experimental import pallas as pl
from jax.experimental.pallas import tpu as pltpu
from jax.experimental.pallas import tpu_sc as plsc

VOCAB = 100000
DIM = 64
BATCH = 4096
LENGTH = 200
N = BATCH * LENGTH

NC = 2
NS = 16
NW = NC * NS
B_PER_W = N // NW
CHUNK = 512
STEPS = B_PER_W // CHUNK

_mesh = plsc.VectorSubcoreMesh(core_axis_name="c", subcore_axis_name="s")


@functools.partial(
    pl.kernel,
    mesh=_mesh,
    out_type=jax.ShapeDtypeStruct((N, DIM), jnp.float32),
    scratch_types=[
        pltpu.VMEM((CHUNK,), jnp.int32),
        pltpu.VMEM((CHUNK, DIM), jnp.float32),
        pltpu.SemaphoreType.DMA,
    ],
    compiler_params=pltpu.CompilerParams(use_tc_tiling_on_sc=False),
)
def _gather_kernel(idx_hbm, table_hbm, out_hbm, idx_v, rows_v, sem):
    wid = lax.axis_index("s") * NC + lax.axis_index("c")
    base = wid * B_PER_W

    def step(i, carry):
        off = base + i * CHUNK
        pltpu.sync_copy(idx_hbm.at[pl.ds(off, CHUNK)], idx_v)
        pltpu.async_copy(table_hbm.at[idx_v], rows_v, sem).wait()
        pltpu.sync_copy(rows_v, out_hbm.at[pl.ds(off, CHUNK)])
        return carry

    lax.fori_loop(0, STEPS, step, 0)


def kernel(wids, wordemb):
    flat = wids.reshape(-1).astype(jnp.int32)
    out = _gather_kernel(flat, wordemb)
    return out.reshape(BATCH, LENGTH, DIM)


# speedup vs baseline: 4.2197x; 1.0666x over previous
import functools

import jax
import jax.numpy as jnp
from jax import lax
from jax.experimental import pallas as pl
from jax.experimental.pallas import tpu as pltpu
from jax.experimental.pallas import tpu_sc as plsc

VOCAB = 100000
DIM = 64
BATCH = 4096
LENGTH = 200
N = BATCH * LENGTH

NC = 2
NS = 16
NW = NC * NS
B_PER_W = N // NW
CHUNK = 512
STEPS = B_PER_W // CHUNK
NBUF = 2
GROUPS = STEPS // NBUF

_mesh = plsc.VectorSubcoreMesh(core_axis_name="c", subcore_axis_name="s")


@functools.partial(
    pl.kernel,
    mesh=_mesh,
    out_type=jax.ShapeDtypeStruct((N, DIM), jnp.float32),
    scratch_types=(
        [pltpu.VMEM((B_PER_W,), jnp.int32)]
        + [pltpu.VMEM((CHUNK, DIM), jnp.float32) for _ in range(NBUF)]
        + [pltpu.SemaphoreType.DMA for _ in range(2 * NBUF)]
    ),
    compiler_params=pltpu.CompilerParams(use_tc_tiling_on_sc=False),
)
def _gather_kernel(idx_hbm, table_hbm, out_hbm, idx_all, *bufs_and_sems):
    rows = bufs_and_sems[:NBUF]
    g_sems = bufs_and_sems[NBUF:2 * NBUF]
    o_sems = bufs_and_sems[2 * NBUF:]

    wid = lax.axis_index("s") * NC + lax.axis_index("c")
    base = wid * B_PER_W

    pltpu.sync_copy(idx_hbm.at[pl.ds(base, B_PER_W)], idx_all)

    def start_gather(b, chunk_i):
        pltpu.make_async_copy(
            table_hbm.at[idx_all.at[pl.ds(chunk_i * CHUNK, CHUNK)]],
            rows[b], g_sems[b]).start()

    def wait_gather(b, chunk_i):
        pltpu.make_async_copy(
            table_hbm.at[idx_all.at[pl.ds(chunk_i * CHUNK, CHUNK)]],
            rows[b], g_sems[b]).wait()

    def out_copy(b, chunk_i):
        return pltpu.make_async_copy(
            rows[b], out_hbm.at[pl.ds(base + chunk_i * CHUNK, CHUNK)],
            o_sems[b])

    for b in range(NBUF):
        start_gather(b, b)

    def group(g, carry):
        for b in range(NBUF):
            i = g * NBUF + b
            wait_gather(b, i)
            out_copy(b, i).start()
        for b in range(NBUF):
            i_next = (g + 1) * NBUF + b

            @pl.when(i_next < STEPS)
            def _():
                out_copy(b, i_next - NBUF).wait()
                start_gather(b, i_next)

        return carry

    lax.fori_loop(0, GROUPS, group, 0)

    for b in range(NBUF):
        out_copy(b, STEPS - NBUF + b).wait()


def kernel(wids, wordemb):
    flat = wids.reshape(-1).astype(jnp.int32)
    out = _gather_kernel(flat, wordemb)
    return out.reshape(BATCH, LENGTH, DIM)


# speedup vs baseline: 4.2322x; 1.0030x over previous
import functools

import jax
import jax.numpy as jnp
from jax import lax
from jax.experimental import pallas as pl
from jax.experimental.pallas import tpu as pltpu
from jax.experimental.pallas import tpu_sc as plsc

VOCAB = 100000
DIM = 64
BATCH = 4096
LENGTH = 200
N = BATCH * LENGTH

NC = 2
NS = 16
NW = NC * NS
B_PER_W = N // NW
CHUNK = 256
STEPS = B_PER_W // CHUNK
NBUF = 4
GROUPS = STEPS // NBUF

_mesh = plsc.VectorSubcoreMesh(core_axis_name="c", subcore_axis_name="s")


@functools.partial(
    pl.kernel,
    mesh=_mesh,
    out_type=jax.ShapeDtypeStruct((N, DIM), jnp.float32),
    scratch_types=(
        [pltpu.VMEM((B_PER_W,), jnp.int32)]
        + [pltpu.VMEM((CHUNK, DIM), jnp.float32) for _ in range(NBUF)]
        + [pltpu.SemaphoreType.DMA for _ in range(2 * NBUF)]
    ),
    compiler_params=pltpu.CompilerParams(use_tc_tiling_on_sc=False),
)
def _gather_kernel(idx_hbm, table_hbm, out_hbm, idx_all, *bufs_and_sems):
    rows = bufs_and_sems[:NBUF]
    g_sems = bufs_and_sems[NBUF:2 * NBUF]
    o_sems = bufs_and_sems[2 * NBUF:]

    wid = lax.axis_index("s") * NC + lax.axis_index("c")
    base = wid * B_PER_W

    pltpu.sync_copy(idx_hbm.at[pl.ds(base, B_PER_W)], idx_all)

    def start_gather(b, chunk_i):
        pltpu.make_async_copy(
            table_hbm.at[idx_all.at[pl.ds(chunk_i * CHUNK, CHUNK)]],
            rows[b], g_sems[b]).start()

    def wait_gather(b, chunk_i):
        pltpu.make_async_copy(
            table_hbm.at[idx_all.at[pl.ds(chunk_i * CHUNK, CHUNK)]],
            rows[b], g_sems[b]).wait()

    def out_copy(b, chunk_i):
        return pltpu.make_async_copy(
            rows[b], out_hbm.at[pl.ds(base + chunk_i * CHUNK, CHUNK)],
            o_sems[b])

    for b in range(NBUF):
        start_gather(b, b)

    def group(g, carry):
        for b in range(NBUF):
            i = g * NBUF + b
            wait_gather(b, i)
            out_copy(b, i).start()
        for b in range(NBUF):
            i_next = (g + 1) * NBUF + b

            @pl.when(i_next < STEPS)
            def _():
                out_copy(b, i_next - NBUF).wait()
                start_gather(b, i_next)

        return carry

    lax.fori_loop(0, GROUPS, group, 0)

    for b in range(NBUF):
        out_copy(b, STEPS - NBUF + b).wait()


def kernel(wids, wordemb):
    flat = wids.reshape(-1).astype(jnp.int32)
    out = _gather_kernel(flat, wordemb)
    return out.reshape(BATCH, LENGTH, DIM)
